# baseline (speedup 1.0000x reference)
import functools

import jax
import jax.numpy as jnp
import numpy as np
from jax import lax
from jax.experimental import pallas as pl
from jax.experimental.pallas import tpu as pltpu
from jax.experimental.pallas import tpu_sc as plsc

HIDDEN = 768
NUM_HEADS = 12
FACTOR = 5


def _proj_kernel(q_ref, k_ref, v_ref, wq_ref, wk_ref, wv_ref,
                 qp_ref, kp_ref, vp_ref):
    qp_ref[...] = jnp.dot(q_ref[...], wq_ref[...],
                          preferred_element_type=jnp.float32)
    kp_ref[...] = jnp.dot(k_ref[...], wk_ref[...],
                          preferred_element_type=jnp.float32)
    vp_ref[...] = jnp.dot(v_ref[...], wv_ref[...],
                          preferred_element_type=jnp.float32)


def _project(q2, k2, v2, Wq, Wk, Wv, tile=512):
    BL, D = q2.shape
    grid = (BL // tile,)
    x_spec = pl.BlockSpec((tile, D), lambda i: (i, 0))
    w_spec = pl.BlockSpec((D, HIDDEN), lambda i: (0, 0))
    o_spec = pl.BlockSpec((tile, HIDDEN), lambda i: (i, 0))
    out_shape = jax.ShapeDtypeStruct((BL, HIDDEN), jnp.float32)
    return pl.pallas_call(
        _proj_kernel,
        grid=grid,
        in_specs=[x_spec, x_spec, x_spec, w_spec, w_spec, w_spec],
        out_specs=[o_spec, o_spec, o_spec],
        out_shape=[out_shape, out_shape, out_shape],
    )(q2, k2, v2, Wq, Wk, Wv)


def _transpose_kernel(src_ref, dst_ref):
    dst_ref[...] = jnp.transpose(src_ref[...], (1, 0))


def _transpose(ctt, L, strip=128):
    return pl.pallas_call(
        _transpose_kernel,
        grid=(L // strip,),
        in_specs=[pl.BlockSpec((strip, L), lambda i: (i, 0))],
        out_specs=pl.BlockSpec((L, strip), lambda i: (0, i)),
        out_shape=jax.ShapeDtypeStruct((L, L), jnp.float32),
    )(ctt)


def _count_kernel_tc(idx_ref, ctt_ref, *, L, u):
    jidx = jax.lax.broadcasted_iota(jnp.int32, (L, L), 1)
    acc = jnp.zeros((L, L), jnp.float32)
    for s in range(u):
        col = idx_ref[:, s:s + 1]
        acc = acc + (jidx == col).astype(jnp.float32)
    ctt_ref[...] = acc


def _build_counts_tc(sample_idx, L, u):
    return pl.pallas_call(
        functools.partial(_count_kernel_tc, L=L, u=u),
        out_shape=jax.ShapeDtypeStruct((L, L), jnp.float32),
    )(sample_idx)


def _build_counts(sample_idx, L, u):
    ROWS = 32
    nchunk = L // ROWS
    ch_per_w = nchunk // 32
    idx_perm = (sample_idx.reshape(nchunk, 2, 16, u)
                .transpose(0, 3, 1, 2).reshape(nchunk * u * ROWS))
    blk_words = ROWS * L
    zeros_blk = jnp.zeros((blk_words,), jnp.float32)

    mesh = plsc.VectorSubcoreMesh(core_axis_name="c", subcore_axis_name="s")

    @functools.partial(
        pl.kernel,
        mesh=mesh,
        out_type=jax.ShapeDtypeStruct((L * L,), jnp.float32),
        scratch_types=[
            pltpu.VMEM((u * ROWS,), jnp.int32),
            pltpu.VMEM((blk_words,), jnp.float32),
        ],
        compiler_params=pltpu.CompilerParams(needs_layout_passes=False),
    )
    def kern(idx_hbm, zeros_hbm, ctt_hbm, idx_v, ct_v):
        wid = lax.axis_index("s") * 2 + lax.axis_index("c")
        ones16 = jnp.ones((16,), jnp.float32)
        for chunk in range(ch_per_w):
            cidx = wid * ch_per_w + chunk
            pltpu.sync_copy(zeros_hbm, ct_v)
            pltpu.sync_copy(idx_hbm.at[pl.ds(cidx * (u * ROWS), u * ROWS)],
                            idx_v)
            for s in range(u):
                for g in range(2):
                    jv = idx_v[pl.ds((s * 2 + g) * 16, 16)]
                    lv = lax.iota(jnp.int32, 16) + g * 16
                    plsc.addupdate_scatter(ct_v, [lv * L + jv], ones16)
            pltpu.sync_copy(ct_v, ctt_hbm.at[pl.ds(cidx * blk_words,
                                                   blk_words)])

    return kern(idx_perm, zeros_blk).reshape(L, L)


def _attn_kernel(qh_ref, kh_ref, vh_ref, ct_ref, out_ref, *, L, u, E, scale,
                 nb, jc):
    ct = ct_ref[...]
    for b in range(nb):
        qh = qh_ref[b]
        kh = kh_ref[b]
        vh = vh_ref[b]

        msum = jnp.zeros((1, L), jnp.float32)
        mmax = jnp.full((1, L), -jnp.inf, jnp.float32)
        for c in range(L // jc):
            kc = kh[c * jc:(c + 1) * jc, :]
            ctc = ct[c * jc:(c + 1) * jc, :]
            stc = jax.lax.dot_general(kc, qh, (((1,), (1,)), ((), ())),
                                      preferred_element_type=jnp.float32)
            msum = msum + jnp.sum(stc * ctc, axis=0, keepdims=True)
            mmax = jnp.maximum(
                mmax, jnp.max(jnp.where(ctc > 0.0, stc, -jnp.inf),
                              axis=0, keepdims=True))
        m = mmax - msum * (1.0 / L)

        lane = jax.lax.broadcasted_iota(jnp.int32, (1, L), 1)
        rows = []
        masks = []
        for _ in range(u):
            cur = jnp.max(m)
            pos = jnp.min(jnp.where(m == cur, lane, L))
            hit = lane == pos
            rows.append(hit.astype(jnp.float32))
            masks.append((lane > pos).astype(jnp.float32))
            m = jnp.where(hit, -jnp.inf, m)
        P = jnp.concatenate(rows, axis=0)
        causal = jnp.concatenate(masks, axis=0)

        qr = jnp.dot(P, qh, preferred_element_type=jnp.float32)
        scores = jax.lax.dot_general(qr, kh, (((1,), (1,)), ((), ())),
                                     preferred_element_type=jnp.float32)
        scores = scores * scale
        scores = jnp.where(causal > 0.0, -jnp.inf, scores)
        smax = jnp.max(scores, axis=1, keepdims=True)
        e = jnp.exp(scores - smax)
        attn = e / jnp.sum(e, axis=1, keepdims=True)
        upd = jnp.dot(attn, vh, preferred_element_type=jnp.float32)

        C = 256
        sub = jax.lax.broadcasted_iota(jnp.int32, (C, C), 0)
        lan = jax.lax.broadcasted_iota(jnp.int32, (C, C), 1)
        tri = (sub >= lan).astype(jnp.float32)
        chunks = []
        carry = jnp.zeros((1, E), jnp.float32)
        for c in range(L // C):
            vc = vh[c * C:(c + 1) * C, :]
            chunks.append(jnp.dot(tri, vc, preferred_element_type=jnp.float32)
                          + carry)
            carry = carry + jnp.sum(vc, axis=0, keepdims=True)
        ctx = jnp.concatenate(chunks, axis=0)

        selcol = jax.lax.dot_general(P, jnp.ones((u, 1), jnp.float32),
                                     (((0,), (0,)), ((), ())),
                                     preferred_element_type=jnp.float32)
        scat = jax.lax.dot_general(P, upd, (((0,), (0,)), ((), ())),
                                   preferred_element_type=jnp.float32)
        out_ref[b] = ctx * (1.0 - selcol) + scat


def _attention(qh, kh, vh, ct, L, u, E, scale, nb=2, jc=512):
    BH = qh.shape[0]
    blk = pl.BlockSpec((nb, L, E), lambda i: (i, 0, 0))
    ct_spec = pl.BlockSpec((L, L), lambda i: (0, 0))
    return pl.pallas_call(
        functools.partial(_attn_kernel, L=L, u=u, E=E, scale=scale,
                          nb=nb, jc=jc),
        grid=(BH // nb,),
        in_specs=[blk, blk, blk, ct_spec],
        out_specs=blk,
        out_shape=jax.ShapeDtypeStruct((BH, L, E), jnp.float32),
    )(qh, kh, vh, ct)


def kernel(q, k, v, Wq, Wk, Wv, sample_idx):
    B, L, D = q.shape
    H = NUM_HEADS
    E = HIDDEN // H
    u = min(FACTOR * int(np.ceil(np.log(L))), L)
    scale = float(1.0 / np.sqrt(HIDDEN // H))

    qp, kp, vp = _project(q.reshape(B * L, D), k.reshape(B * L, D),
                          v.reshape(B * L, D), Wq, Wk, Wv)
    qh = qp.reshape(B * H, L, E)
    kh = kp.reshape(B * H, L, E)
    vh = vp.reshape(B * H, L, E)

    ct = _transpose(_build_counts(sample_idx, L, u), L)
    ctx = _attention(qh, kh, vh, ct, L, u, E, scale)
    return ctx.reshape(B, L, HIDDEN)


# speedup vs baseline: 1.6339x; 1.6339x over previous
import functools

import jax
import jax.numpy as jnp
import numpy as np
from jax import lax
from jax.experimental import pallas as pl
from jax.experimental.pallas import tpu as pltpu
from jax.experimental.pallas import tpu_sc as plsc

HIDDEN = 768
NUM_HEADS = 12
FACTOR = 5


def _proj_kernel(q_ref, k_ref, v_ref, wq_ref, wk_ref, wv_ref,
                 qp_ref, kp_ref, vp_ref):
    qp_ref[...] = jnp.dot(q_ref[...], wq_ref[...],
                          preferred_element_type=jnp.float32)
    kp_ref[...] = jnp.dot(k_ref[...], wk_ref[...],
                          preferred_element_type=jnp.float32)
    vp_ref[...] = jnp.dot(v_ref[...], wv_ref[...],
                          preferred_element_type=jnp.float32)


def _project(q2, k2, v2, Wq, Wk, Wv, tile=512):
    BL, D = q2.shape
    grid = (BL // tile,)
    x_spec = pl.BlockSpec((tile, D), lambda i: (i, 0))
    w_spec = pl.BlockSpec((D, HIDDEN), lambda i: (0, 0))
    o_spec = pl.BlockSpec((tile, HIDDEN), lambda i: (i, 0))
    out_shape = jax.ShapeDtypeStruct((BL, HIDDEN), jnp.float32)
    return pl.pallas_call(
        _proj_kernel,
        grid=grid,
        in_specs=[x_spec, x_spec, x_spec, w_spec, w_spec, w_spec],
        out_specs=[o_spec, o_spec, o_spec],
        out_shape=[out_shape, out_shape, out_shape],
    )(q2, k2, v2, Wq, Wk, Wv)


def _transpose_kernel(src_ref, dst_ref):
    dst_ref[...] = jnp.transpose(src_ref[...], (1, 0))


def _transpose(ctt, L, strip=128):
    return pl.pallas_call(
        _transpose_kernel,
        grid=(L // strip,),
        in_specs=[pl.BlockSpec((strip, L), lambda i: (i, 0))],
        out_specs=pl.BlockSpec((L, strip), lambda i: (0, i)),
        out_shape=jax.ShapeDtypeStruct((L, L), jnp.float32),
    )(ctt)


def _count_kernel_tc(idx_ref, ctt_ref, *, L, u):
    jidx = jax.lax.broadcasted_iota(jnp.int32, (L, L), 1)
    acc = jnp.zeros((L, L), jnp.float32)
    for s in range(u):
        col = idx_ref[:, s:s + 1]
        acc = acc + (jidx == col).astype(jnp.float32)
    ctt_ref[...] = acc


def _build_counts_tc(sample_idx, L, u):
    return pl.pallas_call(
        functools.partial(_count_kernel_tc, L=L, u=u),
        out_shape=jax.ShapeDtypeStruct((L, L), jnp.float32),
    )(sample_idx)


def _build_counts(sample_idx, L, u):
    ROWS = 32
    nchunk = L // ROWS
    ch_per_w = nchunk // 32
    idx_perm = (sample_idx.reshape(nchunk, 2, 16, u)
                .transpose(0, 3, 1, 2).reshape(nchunk * u * ROWS))
    blk_words = ROWS * L
    zeros_blk = jnp.zeros((blk_words,), jnp.float32)

    mesh = plsc.VectorSubcoreMesh(core_axis_name="c", subcore_axis_name="s")

    @functools.partial(
        pl.kernel,
        mesh=mesh,
        out_type=jax.ShapeDtypeStruct((L * L,), jnp.float32),
        scratch_types=[
            pltpu.VMEM((u * ROWS,), jnp.int32),
            pltpu.VMEM((blk_words,), jnp.float32),
        ],
        compiler_params=pltpu.CompilerParams(needs_layout_passes=False),
    )
    def kern(idx_hbm, zeros_hbm, ctt_hbm, idx_v, ct_v):
        wid = lax.axis_index("s") * 2 + lax.axis_index("c")
        ones16 = jnp.ones((16,), jnp.float32)
        for chunk in range(ch_per_w):
            cidx = wid * ch_per_w + chunk
            pltpu.sync_copy(zeros_hbm, ct_v)
            pltpu.sync_copy(idx_hbm.at[pl.ds(cidx * (u * ROWS), u * ROWS)],
                            idx_v)
            for s in range(u):
                for g in range(2):
                    jv = idx_v[pl.ds((s * 2 + g) * 16, 16)]
                    lv = lax.iota(jnp.int32, 16) + g * 16
                    plsc.addupdate_scatter(ct_v, [lv * L + jv], ones16)
            pltpu.sync_copy(ct_v, ctt_hbm.at[pl.ds(cidx * blk_words,
                                                   blk_words)])

    return kern(idx_perm, zeros_blk).reshape(L, L)


def _attn_kernel(qh_ref, kh_ref, vh_ref, ct_ref, out_ref, *, L, u, E, scale,
                 nb, jc):
    ct = ct_ref[...]

    msum = [jnp.zeros((1, L), jnp.float32) for _ in range(nb)]
    mmax = [jnp.full((1, L), -jnp.inf, jnp.float32) for _ in range(nb)]
    for c in range(L // jc):
        ctc = ct[c * jc:(c + 1) * jc, :]
        biasc = jnp.where(ctc > 0.0, 0.0, -jnp.inf)
        for b in range(nb):
            kc = kh_ref[b, c * jc:(c + 1) * jc, :]
            stc = jax.lax.dot_general(kc, qh_ref[b],
                                      (((1,), (1,)), ((), ())),
                                      preferred_element_type=jnp.float32)
            msum[b] = msum[b] + jnp.sum(stc * ctc, axis=0, keepdims=True)
            mmax[b] = jnp.maximum(
                mmax[b], jnp.max(stc + biasc, axis=0, keepdims=True))
    mm = jnp.concatenate(
        [mmax[b] - msum[b] * (1.0 / L) for b in range(nb)], axis=0)

    lane = jax.lax.broadcasted_iota(jnp.int32, (nb, L), 1)
    hits = []
    gts = []
    for _ in range(u):
        cur = jnp.max(mm, axis=1, keepdims=True)
        pos = jnp.min(jnp.where(mm == cur, lane, L),
                      axis=1, keepdims=True)
        hit = lane == pos
        hits.append(hit.astype(jnp.float32))
        gts.append((lane > pos).astype(jnp.float32))
        mm = jnp.where(hit, -jnp.inf, mm)

    C = 256
    sub = jax.lax.broadcasted_iota(jnp.int32, (C, C), 0)
    lan = jax.lax.broadcasted_iota(jnp.int32, (C, C), 1)
    tri = (sub >= lan).astype(jnp.float32)

    for b in range(nb):
        qh = qh_ref[b]
        kh = kh_ref[b]
        vh = vh_ref[b]
        P = jnp.concatenate([h[b:b + 1, :] for h in hits], axis=0)
        causal = jnp.concatenate([g[b:b + 1, :] for g in gts], axis=0)

        qr = jnp.dot(P, qh, preferred_element_type=jnp.float32)
        scores = jax.lax.dot_general(qr, kh, (((1,), (1,)), ((), ())),
                                     preferred_element_type=jnp.float32)
        scores = scores * scale
        scores = jnp.where(causal > 0.0, -jnp.inf, scores)
        smax = jnp.max(scores, axis=1, keepdims=True)
        e = jnp.exp(scores - smax)
        attn = e / jnp.sum(e, axis=1, keepdims=True)
        upd = jnp.dot(attn, vh, preferred_element_type=jnp.float32)

        chunks = []
        carry = jnp.zeros((1, E), jnp.float32)
        for c in range(L // C):
            vc = vh[c * C:(c + 1) * C, :]
            chunks.append(jnp.dot(tri, vc, preferred_element_type=jnp.float32)
                          + carry)
            carry = carry + jnp.sum(vc, axis=0, keepdims=True)
        ctx = jnp.concatenate(chunks, axis=0)

        selcol = jax.lax.dot_general(P, jnp.ones((u, 1), jnp.float32),
                                     (((0,), (0,)), ((), ())),
                                     preferred_element_type=jnp.float32)
        scat = jax.lax.dot_general(P, upd, (((0,), (0,)), ((), ())),
                                   preferred_element_type=jnp.float32)
        out_ref[b] = ctx * (1.0 - selcol) + scat


def _attention(qh, kh, vh, ct, L, u, E, scale, nb=4, jc=256):
    BH = qh.shape[0]
    blk = pl.BlockSpec((nb, L, E), lambda i: (i, 0, 0))
    ct_spec = pl.BlockSpec((L, L), lambda i: (0, 0))
    return pl.pallas_call(
        functools.partial(_attn_kernel, L=L, u=u, E=E, scale=scale,
                          nb=nb, jc=jc),
        grid=(BH // nb,),
        in_specs=[blk, blk, blk, ct_spec],
        out_specs=blk,
        out_shape=jax.ShapeDtypeStruct((BH, L, E), jnp.float32),
    )(qh, kh, vh, ct)


def kernel(q, k, v, Wq, Wk, Wv, sample_idx):
    B, L, D = q.shape
    H = NUM_HEADS
    E = HIDDEN // H
    u = min(FACTOR * int(np.ceil(np.log(L))), L)
    scale = float(1.0 / np.sqrt(HIDDEN // H))

    qp, kp, vp = _project(q.reshape(B * L, D), k.reshape(B * L, D),
                          v.reshape(B * L, D), Wq, Wk, Wv)
    qh = qp.reshape(B * H, L, E)
    kh = kp.reshape(B * H, L, E)
    vh = vp.reshape(B * H, L, E)

    ct = _transpose(_build_counts(sample_idx, L, u), L)
    ctx = _attention(qh, kh, vh, ct, L, u, E, scale)
    return ctx.reshape(B, L, HIDDEN)


# speedup vs baseline: 1.6343x; 1.0003x over previous
import functools

import jax
import jax.numpy as jnp
import numpy as np
from jax import lax
from jax.experimental import pallas as pl
from jax.experimental.pallas import tpu as pltpu
from jax.experimental.pallas import tpu_sc as plsc

HIDDEN = 768
NUM_HEADS = 12
FACTOR = 5


def _proj_kernel(q_ref, k_ref, v_ref, wq_ref, wk_ref, wv_ref,
                 qp_ref, kp_ref, vp_ref):
    qp_ref[...] = jnp.dot(q_ref[...], wq_ref[...],
                          preferred_element_type=jnp.float32)
    kp_ref[...] = jnp.dot(k_ref[...], wk_ref[...],
                          preferred_element_type=jnp.float32)
    vp_ref[...] = jnp.dot(v_ref[...], wv_ref[...],
                          preferred_element_type=jnp.float32)


def _project(q2, k2, v2, Wq, Wk, Wv, tile=512):
    BL, D = q2.shape
    grid = (BL // tile,)
    x_spec = pl.BlockSpec((tile, D), lambda i: (i, 0))
    w_spec = pl.BlockSpec((D, HIDDEN), lambda i: (0, 0))
    o_spec = pl.BlockSpec((tile, HIDDEN), lambda i: (i, 0))
    out_shape = jax.ShapeDtypeStruct((BL, HIDDEN), jnp.float32)
    return pl.pallas_call(
        _proj_kernel,
        grid=grid,
        in_specs=[x_spec, x_spec, x_spec, w_spec, w_spec, w_spec],
        out_specs=[o_spec, o_spec, o_spec],
        out_shape=[out_shape, out_shape, out_shape],
    )(q2, k2, v2, Wq, Wk, Wv)


def _transpose_kernel(src_ref, dst_ref):
    dst_ref[...] = jnp.transpose(src_ref[...], (1, 0))


def _transpose(ctt, L, strip=128):
    return pl.pallas_call(
        _transpose_kernel,
        grid=(L // strip,),
        in_specs=[pl.BlockSpec((strip, L), lambda i: (i, 0))],
        out_specs=pl.BlockSpec((L, strip), lambda i: (0, i)),
        out_shape=jax.ShapeDtypeStruct((L, L), jnp.float32),
    )(ctt)


def _count_kernel_tc(idx_ref, ctt_ref, *, L, u):
    jidx = jax.lax.broadcasted_iota(jnp.int32, (L, L), 1)
    acc = jnp.zeros((L, L), jnp.float32)
    for s in range(u):
        col = idx_ref[:, s:s + 1]
        acc = acc + (jidx == col).astype(jnp.float32)
    ctt_ref[...] = acc


def _build_counts_tc(sample_idx, L, u):
    return pl.pallas_call(
        functools.partial(_count_kernel_tc, L=L, u=u),
        out_shape=jax.ShapeDtypeStruct((L, L), jnp.float32),
    )(sample_idx)


def _build_counts(sample_idx, L, u):
    ROWS = 32
    nchunk = L // ROWS
    ch_per_w = nchunk // 32
    idx_perm = (sample_idx.reshape(nchunk, 2, 16, u)
                .transpose(0, 3, 1, 2).reshape(nchunk * u * ROWS))
    blk_words = ROWS * L
    zeros_blk = jnp.zeros((blk_words,), jnp.float32)

    mesh = plsc.VectorSubcoreMesh(core_axis_name="c", subcore_axis_name="s")

    @functools.partial(
        pl.kernel,
        mesh=mesh,
        out_type=jax.ShapeDtypeStruct((L * L,), jnp.float32),
        scratch_types=[
            pltpu.VMEM((u * ROWS,), jnp.int32),
            pltpu.VMEM((blk_words,), jnp.float32),
        ],
        compiler_params=pltpu.CompilerParams(needs_layout_passes=False),
    )
    def kern(idx_hbm, zeros_hbm, ctt_hbm, idx_v, ct_v):
        wid = lax.axis_index("s") * 2 + lax.axis_index("c")
        ones16 = jnp.ones((16,), jnp.float32)
        for chunk in range(ch_per_w):
            cidx = wid * ch_per_w + chunk
            pltpu.sync_copy(zeros_hbm, ct_v)
            pltpu.sync_copy(idx_hbm.at[pl.ds(cidx * (u * ROWS), u * ROWS)],
                            idx_v)
            for s in range(u):
                for g in range(2):
                    jv = idx_v[pl.ds((s * 2 + g) * 16, 16)]
                    lv = lax.iota(jnp.int32, 16) + g * 16
                    plsc.addupdate_scatter(ct_v, [lv * L + jv], ones16)
            pltpu.sync_copy(ct_v, ctt_hbm.at[pl.ds(cidx * blk_words,
                                                   blk_words)])

    return kern(idx_perm, zeros_blk).reshape(L, L)


def _attn_kernel(qh_ref, kh_ref, vh_ref, ct_ref, out_ref, *, L, u, E, scale,
                 nb, jc):
    ct = ct_ref[...]

    msum = [jnp.zeros((1, L), jnp.float32) for _ in range(nb)]
    mmax = [jnp.full((1, L), -jnp.inf, jnp.float32) for _ in range(nb)]
    for c in range(L // jc):
        ctc = ct[c * jc:(c + 1) * jc, :]
        biasc = jnp.where(ctc > 0.0, 0.0, -jnp.inf)
        for b in range(nb):
            kc = kh_ref[b, c * jc:(c + 1) * jc, :]
            stc = jax.lax.dot_general(kc, qh_ref[b],
                                      (((1,), (1,)), ((), ())),
                                      preferred_element_type=jnp.float32)
            msum[b] = msum[b] + jnp.sum(stc * ctc, axis=0, keepdims=True)
            mmax[b] = jnp.maximum(
                mmax[b], jnp.max(stc + biasc, axis=0, keepdims=True))
    mm = jnp.concatenate(
        [mmax[b] - msum[b] * (1.0 / L) for b in range(nb)], axis=0)

    lane = jax.lax.broadcasted_iota(jnp.int32, (nb, L), 1)
    poss = []
    for _ in range(u):
        cur = jnp.max(mm, axis=1, keepdims=True)
        pos = jnp.min(jnp.where(mm == cur, lane, L),
                      axis=1, keepdims=True)
        poss.append(pos)
        mm = jnp.where(lane == pos, -jnp.inf, mm)
    posr = jnp.transpose(jnp.concatenate(poss, axis=1), (1, 0))
    lane_u = jax.lax.broadcasted_iota(jnp.int32, (u, L), 1)

    C = 256
    sub = jax.lax.broadcasted_iota(jnp.int32, (C, C), 0)
    lan = jax.lax.broadcasted_iota(jnp.int32, (C, C), 1)
    tri = (sub >= lan).astype(jnp.float32)

    for b in range(nb):
        qh = qh_ref[b]
        kh = kh_ref[b]
        vh = vh_ref[b]
        pcol = posr[:, b:b + 1]
        P = (lane_u == pcol).astype(jnp.float32)
        causal = (lane_u > pcol).astype(jnp.float32)

        qr = jnp.dot(P, qh, preferred_element_type=jnp.float32)
        scores = jax.lax.dot_general(qr, kh, (((1,), (1,)), ((), ())),
                                     preferred_element_type=jnp.float32)
        scores = scores * scale
        scores = jnp.where(causal > 0.0, -jnp.inf, scores)
        smax = jnp.max(scores, axis=1, keepdims=True)
        e = jnp.exp(scores - smax)
        attn = e / jnp.sum(e, axis=1, keepdims=True)
        upd = jnp.dot(attn, vh, preferred_element_type=jnp.float32)

        chunks = []
        carry = jnp.zeros((1, E), jnp.float32)
        for c in range(L // C):
            vc = vh[c * C:(c + 1) * C, :]
            chunks.append(jnp.dot(tri, vc, preferred_element_type=jnp.float32)
                          + carry)
            carry = carry + jnp.sum(vc, axis=0, keepdims=True)
        ctx = jnp.concatenate(chunks, axis=0)

        selcol = jax.lax.dot_general(P, jnp.ones((u, 1), jnp.float32),
                                     (((0,), (0,)), ((), ())),
                                     preferred_element_type=jnp.float32)
        scat = jax.lax.dot_general(P, upd, (((0,), (0,)), ((), ())),
                                   preferred_element_type=jnp.float32)
        out_ref[b] = ctx * (1.0 - selcol) + scat


def _attention(qh, kh, vh, ct, L, u, E, scale, nb=4, jc=256):
    BH = qh.shape[0]
    blk = pl.BlockSpec((nb, L, E), lambda i: (i, 0, 0))
    ct_spec = pl.BlockSpec((L, L), lambda i: (0, 0))
    return pl.pallas_call(
        functools.partial(_attn_kernel, L=L, u=u, E=E, scale=scale,
                          nb=nb, jc=jc),
        grid=(BH // nb,),
        in_specs=[blk, blk, blk, ct_spec],
        out_specs=blk,
        out_shape=jax.ShapeDtypeStruct((BH, L, E), jnp.float32),
    )(qh, kh, vh, ct)


def kernel(q, k, v, Wq, Wk, Wv, sample_idx):
    B, L, D = q.shape
    H = NUM_HEADS
    E = HIDDEN // H
    u = min(FACTOR * int(np.ceil(np.log(L))), L)
    scale = float(1.0 / np.sqrt(HIDDEN // H))

    qp, kp, vp = _project(q.reshape(B * L, D), k.reshape(B * L, D),
                          v.reshape(B * L, D), Wq, Wk, Wv)
    qh = qp.reshape(B * H, L, E)
    kh = kp.reshape(B * H, L, E)
    vh = vp.reshape(B * H, L, E)

    ct = _transpose(_build_counts(sample_idx, L, u), L)
    ctx = _attention(qh, kh, vh, ct, L, u, E, scale)
    return ctx.reshape(B, L, HIDDEN)


# speedup vs baseline: 1.6474x; 1.0080x over previous
import functools

import jax
import jax.numpy as jnp
import numpy as np
from jax import lax
from jax.experimental import pallas as pl
from jax.experimental.pallas import tpu as pltpu
from jax.experimental.pallas import tpu_sc as plsc

HIDDEN = 768
NUM_HEADS = 12
FACTOR = 5


def _proj_kernel(q_ref, k_ref, v_ref, wq_ref, wk_ref, wv_ref,
                 qp_ref, kp_ref, vp_ref):
    qp_ref[...] = jnp.dot(q_ref[...], wq_ref[...],
                          preferred_element_type=jnp.float32)
    kp_ref[...] = jnp.dot(k_ref[...], wk_ref[...],
                          preferred_element_type=jnp.float32)
    vp_ref[...] = jnp.dot(v_ref[...], wv_ref[...],
                          preferred_element_type=jnp.float32)


def _project(q2, k2, v2, Wq, Wk, Wv, tile=512):
    BL, D = q2.shape
    grid = (BL // tile,)
    x_spec = pl.BlockSpec((tile, D), lambda i: (i, 0))
    w_spec = pl.BlockSpec((D, HIDDEN), lambda i: (0, 0))
    o_spec = pl.BlockSpec((tile, HIDDEN), lambda i: (i, 0))
    out_shape = jax.ShapeDtypeStruct((BL, HIDDEN), jnp.float32)
    return pl.pallas_call(
        _proj_kernel,
        grid=grid,
        in_specs=[x_spec, x_spec, x_spec, w_spec, w_spec, w_spec],
        out_specs=[o_spec, o_spec, o_spec],
        out_shape=[out_shape, out_shape, out_shape],
    )(q2, k2, v2, Wq, Wk, Wv)


def _transpose_kernel(src_ref, dst_ref):
    dst_ref[...] = jnp.transpose(src_ref[...], (1, 0))


def _transpose(ctt, L, strip=128):
    return pl.pallas_call(
        _transpose_kernel,
        grid=(L // strip,),
        in_specs=[pl.BlockSpec((strip, L), lambda i: (i, 0))],
        out_specs=pl.BlockSpec((L, strip), lambda i: (0, i)),
        out_shape=jax.ShapeDtypeStruct((L, L), jnp.float32),
    )(ctt)


def _count_kernel_tc(idx_ref, ctt_ref, *, L, u):
    jidx = jax.lax.broadcasted_iota(jnp.int32, (L, L), 1)
    acc = jnp.zeros((L, L), jnp.float32)
    for s in range(u):
        col = idx_ref[:, s:s + 1]
        acc = acc + (jidx == col).astype(jnp.float32)
    ctt_ref[...] = acc


def _build_counts_tc(sample_idx, L, u):
    return pl.pallas_call(
        functools.partial(_count_kernel_tc, L=L, u=u),
        out_shape=jax.ShapeDtypeStruct((L, L), jnp.float32),
    )(sample_idx)


def _build_counts(sample_idx, L, u):
    ROWS = 32
    nchunk = L // ROWS
    ch_per_w = nchunk // 32
    idx_perm = (sample_idx.reshape(nchunk, 2, 16, u)
                .transpose(0, 3, 1, 2).reshape(nchunk * u * ROWS))
    blk_words = ROWS * L
    zeros_blk = jnp.zeros((blk_words,), jnp.float32)

    mesh = plsc.VectorSubcoreMesh(core_axis_name="c", subcore_axis_name="s")

    @functools.partial(
        pl.kernel,
        mesh=mesh,
        out_type=jax.ShapeDtypeStruct((L * L,), jnp.float32),
        scratch_types=[
            pltpu.VMEM((u * ROWS,), jnp.int32),
            pltpu.VMEM((blk_words,), jnp.float32),
        ],
        compiler_params=pltpu.CompilerParams(needs_layout_passes=False),
    )
    def kern(idx_hbm, zeros_hbm, ctt_hbm, idx_v, ct_v):
        wid = lax.axis_index("s") * 2 + lax.axis_index("c")
        ones16 = jnp.ones((16,), jnp.float32)
        for chunk in range(ch_per_w):
            cidx = wid * ch_per_w + chunk
            pltpu.sync_copy(zeros_hbm, ct_v)
            pltpu.sync_copy(idx_hbm.at[pl.ds(cidx * (u * ROWS), u * ROWS)],
                            idx_v)
            for s in range(u):
                for g in range(2):
                    jv = idx_v[pl.ds((s * 2 + g) * 16, 16)]
                    lv = lax.iota(jnp.int32, 16) + g * 16
                    plsc.addupdate_scatter(ct_v, [lv * L + jv], ones16)
            pltpu.sync_copy(ct_v, ctt_hbm.at[pl.ds(cidx * blk_words,
                                                   blk_words)])

    return kern(idx_perm, zeros_blk).reshape(L, L)


def _attn_kernel(qh_ref, kh_ref, vh_ref, ct_ref, out_ref, *, L, u, E, scale,
                 nb, jc):
    ct = ct_ref[...]

    msum = [jnp.zeros((1, L), jnp.float32) for _ in range(nb)]
    mmax = [jnp.full((1, L), -jnp.inf, jnp.float32) for _ in range(nb)]
    for c in range(L // jc):
        ctc = ct[c * jc:(c + 1) * jc, :]
        biasc = jnp.where(ctc > 0.0, 0.0, -jnp.inf)
        for b in range(nb):
            kc = kh_ref[b, c * jc:(c + 1) * jc, :]
            stc = jax.lax.dot_general(kc, qh_ref[b],
                                      (((1,), (1,)), ((), ())),
                                      preferred_element_type=jnp.float32)
            msum[b] = msum[b] + jnp.sum(stc * ctc, axis=0, keepdims=True)
            mmax[b] = jnp.maximum(
                mmax[b], jnp.max(stc + biasc, axis=0, keepdims=True))
    mm = jnp.concatenate(
        [mmax[b] - msum[b] * (1.0 / L) for b in range(nb)], axis=0)

    C = 256
    sub = jax.lax.broadcasted_iota(jnp.int32, (C, C), 0)
    lan = jax.lax.broadcasted_iota(jnp.int32, (C, C), 1)
    tri = (sub >= lan).astype(jnp.float32)
    ctxs = []
    for b in range(nb):
        vh = vh_ref[b]
        chunks = []
        carry = jnp.zeros((1, E), jnp.float32)
        for c in range(L // C):
            vc = vh[c * C:(c + 1) * C, :]
            chunks.append(jnp.dot(tri, vc, preferred_element_type=jnp.float32)
                          + carry)
            carry = carry + jnp.sum(vc, axis=0, keepdims=True)
        ctxs.append(jnp.concatenate(chunks, axis=0))

    lane = jax.lax.broadcasted_iota(jnp.int32, (nb, L), 1)
    poss = []
    for _ in range(u):
        cur = jnp.max(mm, axis=1, keepdims=True)
        pos = jnp.min(jnp.where(mm == cur, lane, L),
                      axis=1, keepdims=True)
        poss.append(pos)
        mm = jnp.where(lane == pos, -jnp.inf, mm)
    posr = jnp.transpose(jnp.concatenate(poss, axis=1), (1, 0))
    lane_u = jax.lax.broadcasted_iota(jnp.int32, (u, L), 1)

    for b in range(nb):
        qh = qh_ref[b]
        kh = kh_ref[b]
        vh = vh_ref[b]
        pcol = posr[:, b:b + 1]
        P = (lane_u == pcol).astype(jnp.float32)
        causal = (lane_u > pcol).astype(jnp.float32)

        qr = jnp.dot(P, qh, preferred_element_type=jnp.float32)
        scores = jax.lax.dot_general(qr, kh, (((1,), (1,)), ((), ())),
                                     preferred_element_type=jnp.float32)
        scores = scores * scale
        scores = jnp.where(causal > 0.0, -jnp.inf, scores)
        smax = jnp.max(scores, axis=1, keepdims=True)
        e = jnp.exp(scores - smax)
        attn = e / jnp.sum(e, axis=1, keepdims=True)
        upd = jnp.dot(attn, vh, preferred_element_type=jnp.float32)
        ctx = ctxs[b]

        selcol = jax.lax.dot_general(P, jnp.ones((u, 1), jnp.float32),
                                     (((0,), (0,)), ((), ())),
                                     preferred_element_type=jnp.float32)
        scat = jax.lax.dot_general(P, upd, (((0,), (0,)), ((), ())),
                                   preferred_element_type=jnp.float32)
        out_ref[b] = ctx * (1.0 - selcol) + scat


def _attention(qh, kh, vh, ct, L, u, E, scale, nb=4, jc=256):
    BH = qh.shape[0]
    blk = pl.BlockSpec((nb, L, E), lambda i: (i, 0, 0))
    ct_spec = pl.BlockSpec((L, L), lambda i: (0, 0))
    return pl.pallas_call(
        functools.partial(_attn_kernel, L=L, u=u, E=E, scale=scale,
                          nb=nb, jc=jc),
        grid=(BH // nb,),
        in_specs=[blk, blk, blk, ct_spec],
        out_specs=blk,
        out_shape=jax.ShapeDtypeStruct((BH, L, E), jnp.float32),
    )(qh, kh, vh, ct)


def kernel(q, k, v, Wq, Wk, Wv, sample_idx):
    B, L, D = q.shape
    H = NUM_HEADS
    E = HIDDEN // H
    u = min(FACTOR * int(np.ceil(np.log(L))), L)
    scale = float(1.0 / np.sqrt(HIDDEN // H))

    qp, kp, vp = _project(q.reshape(B * L, D), k.reshape(B * L, D),
                          v.reshape(B * L, D), Wq, Wk, Wv)
    qh = qp.reshape(B * H, L, E)
    kh = kp.reshape(B * H, L, E)
    vh = vp.reshape(B * H, L, E)

    ct = _transpose(_build_counts(sample_idx, L, u), L)
    ctx = _attention(qh, kh, vh, ct, L, u, E, scale)
    return ctx.reshape(B, L, HIDDEN)


# speedup vs baseline: 1.6705x; 1.0140x over previous
import functools

import jax
import jax.numpy as jnp
import numpy as np
from jax import lax
from jax.experimental import pallas as pl
from jax.experimental.pallas import tpu as pltpu
from jax.experimental.pallas import tpu_sc as plsc

HIDDEN = 768
NUM_HEADS = 12
FACTOR = 5


def _proj_kernel(q_ref, k_ref, v_ref, wq_ref, wk_ref, wv_ref,
                 qp_ref, kp_ref, vp_ref):
    qp_ref[...] = jnp.dot(q_ref[...], wq_ref[...],
                          preferred_element_type=jnp.float32)
    kp_ref[...] = jnp.dot(k_ref[...], wk_ref[...],
                          preferred_element_type=jnp.float32)
    vp_ref[...] = jnp.dot(v_ref[...], wv_ref[...],
                          preferred_element_type=jnp.float32)


def _project(q2, k2, v2, Wq, Wk, Wv, tile=512):
    BL, D = q2.shape
    grid = (BL // tile,)
    x_spec = pl.BlockSpec((tile, D), lambda i: (i, 0))
    w_spec = pl.BlockSpec((D, HIDDEN), lambda i: (0, 0))
    o_spec = pl.BlockSpec((tile, HIDDEN), lambda i: (i, 0))
    out_shape = jax.ShapeDtypeStruct((BL, HIDDEN), jnp.float32)
    return pl.pallas_call(
        _proj_kernel,
        grid=grid,
        in_specs=[x_spec, x_spec, x_spec, w_spec, w_spec, w_spec],
        out_specs=[o_spec, o_spec, o_spec],
        out_shape=[out_shape, out_shape, out_shape],
    )(q2, k2, v2, Wq, Wk, Wv)


def _transpose_kernel(src_ref, dst_ref):
    dst_ref[...] = jnp.transpose(src_ref[...], (1, 0))


def _transpose(ctt, L, strip=128):
    return pl.pallas_call(
        _transpose_kernel,
        grid=(L // strip,),
        in_specs=[pl.BlockSpec((strip, L), lambda i: (i, 0))],
        out_specs=pl.BlockSpec((L, strip), lambda i: (0, i)),
        out_shape=jax.ShapeDtypeStruct((L, L), jnp.float32),
    )(ctt)


def _count_kernel_tc(idx_ref, ctt_ref, *, L, u):
    jidx = jax.lax.broadcasted_iota(jnp.int32, (L, L), 1)
    acc = jnp.zeros((L, L), jnp.float32)
    for s in range(u):
        col = idx_ref[:, s:s + 1]
        acc = acc + (jidx == col).astype(jnp.float32)
    ctt_ref[...] = acc


def _build_counts_tc(sample_idx, L, u):
    return pl.pallas_call(
        functools.partial(_count_kernel_tc, L=L, u=u),
        out_shape=jax.ShapeDtypeStruct((L, L), jnp.float32),
    )(sample_idx)


def _build_counts(sample_idx, L, u):
    ROWS = 32
    nchunk = L // ROWS
    ch_per_w = nchunk // 32
    idx_perm = (sample_idx.reshape(nchunk, 2, 16, u)
                .transpose(0, 3, 1, 2).reshape(nchunk * u * ROWS))
    blk_words = ROWS * L
    zeros_blk = jnp.zeros((blk_words,), jnp.float32)

    mesh = plsc.VectorSubcoreMesh(core_axis_name="c", subcore_axis_name="s")

    @functools.partial(
        pl.kernel,
        mesh=mesh,
        out_type=jax.ShapeDtypeStruct((L * L,), jnp.float32),
        scratch_types=[
            pltpu.VMEM((u * ROWS,), jnp.int32),
            pltpu.VMEM((blk_words,), jnp.float32),
        ],
        compiler_params=pltpu.CompilerParams(needs_layout_passes=False),
    )
    def kern(idx_hbm, zeros_hbm, ctt_hbm, idx_v, ct_v):
        wid = lax.axis_index("s") * 2 + lax.axis_index("c")
        ones16 = jnp.ones((16,), jnp.float32)
        for chunk in range(ch_per_w):
            cidx = wid * ch_per_w + chunk
            pltpu.sync_copy(zeros_hbm, ct_v)
            pltpu.sync_copy(idx_hbm.at[pl.ds(cidx * (u * ROWS), u * ROWS)],
                            idx_v)
            for s in range(u):
                for g in range(2):
                    jv = idx_v[pl.ds((s * 2 + g) * 16, 16)]
                    lv = lax.iota(jnp.int32, 16) + g * 16
                    plsc.addupdate_scatter(ct_v, [lv * L + jv], ones16)
            pltpu.sync_copy(ct_v, ctt_hbm.at[pl.ds(cidx * blk_words,
                                                   blk_words)])

    return kern(idx_perm, zeros_blk).reshape(L, L)


def _attn_kernel(qh_ref, kh_ref, vh_ref, ct_ref, out_ref, *, L, u, E, scale,
                 nb, jc):
    ct = ct_ref[...]

    msum = [jnp.zeros((1, L), jnp.float32) for _ in range(nb)]
    mmax = [jnp.full((1, L), -jnp.inf, jnp.float32) for _ in range(nb)]
    for c in range(L // jc):
        ctc = ct[c * jc:(c + 1) * jc, :]
        biasc = jnp.where(ctc > 0.0, 0.0, -jnp.inf)
        for b in range(nb):
            kc = kh_ref[b, c * jc:(c + 1) * jc, :]
            stc = jax.lax.dot_general(kc, qh_ref[b],
                                      (((1,), (1,)), ((), ())),
                                      preferred_element_type=jnp.float32)
            msum[b] = msum[b] + jnp.sum(stc * ctc, axis=0, keepdims=True)
            mmax[b] = jnp.maximum(
                mmax[b], jnp.max(stc + biasc, axis=0, keepdims=True))
    mm = jnp.concatenate(
        [mmax[b] - msum[b] * (1.0 / L) for b in range(nb)], axis=0)

    C = 256
    sub = jax.lax.broadcasted_iota(jnp.int32, (C, C), 0)
    lan = jax.lax.broadcasted_iota(jnp.int32, (C, C), 1)
    tri = (sub >= lan).astype(jnp.float32)
    ctxs = []
    for b in range(nb):
        vh = vh_ref[b]
        chunks = []
        carry = jnp.zeros((1, E), jnp.float32)
        for c in range(L // C):
            vc = vh[c * C:(c + 1) * C, :]
            chunks.append(jnp.dot(tri, vc, preferred_element_type=jnp.float32)
                          + carry)
            carry = carry + jnp.sum(vc, axis=0, keepdims=True)
        ctxs.append(jnp.concatenate(chunks, axis=0))

    lane = jax.lax.broadcasted_iota(jnp.int32, (nb, L), 1)
    poss = []
    for _ in range(u):
        cur = jnp.max(mm, axis=1, keepdims=True)
        pos = jnp.min(jnp.where(mm == cur, lane, L),
                      axis=1, keepdims=True)
        poss.append(pos)
        mm = jnp.where(lane == pos, -jnp.inf, mm)
    posr = jnp.transpose(jnp.concatenate(poss, axis=1), (1, 0))
    lane_u = jax.lax.broadcasted_iota(jnp.int32, (u, L), 1)

    for b in range(nb):
        qh = qh_ref[b]
        kh = kh_ref[b]
        vh = vh_ref[b]
        pcol = posr[:, b:b + 1]
        P = (lane_u == pcol).astype(jnp.float32)
        causal = (lane_u > pcol).astype(jnp.float32)

        qr = jnp.dot(P, qh, preferred_element_type=jnp.float32)
        scores = jax.lax.dot_general(qr, kh, (((1,), (1,)), ((), ())),
                                     preferred_element_type=jnp.float32)
        scores = scores * scale
        scores = jnp.where(causal > 0.0, -jnp.inf, scores)
        smax = jnp.max(scores, axis=1, keepdims=True)
        e = jnp.exp(scores - smax)
        attn = e / jnp.sum(e, axis=1, keepdims=True)
        upd = jnp.dot(attn, vh, preferred_element_type=jnp.float32)
        ctx = ctxs[b]

        selcol = jax.lax.dot_general(P, jnp.ones((u, 1), jnp.float32),
                                     (((0,), (0,)), ((), ())),
                                     preferred_element_type=jnp.float32)
        scat = jax.lax.dot_general(P, upd, (((0,), (0,)), ((), ())),
                                   preferred_element_type=jnp.float32)
        out_ref[b] = ctx * (1.0 - selcol) + scat


def _attention(qh, kh, vh, ct, L, u, E, scale, nb=4, jc=512):
    BH = qh.shape[0]
    blk = pl.BlockSpec((nb, L, E), lambda i: (i, 0, 0))
    ct_spec = pl.BlockSpec((L, L), lambda i: (0, 0))
    return pl.pallas_call(
        functools.partial(_attn_kernel, L=L, u=u, E=E, scale=scale,
                          nb=nb, jc=jc),
        grid=(BH // nb,),
        in_specs=[blk, blk, blk, ct_spec],
        out_specs=blk,
        out_shape=jax.ShapeDtypeStruct((BH, L, E), jnp.float32),
    )(qh, kh, vh, ct)


def kernel(q, k, v, Wq, Wk, Wv, sample_idx):
    B, L, D = q.shape
    H = NUM_HEADS
    E = HIDDEN // H
    u = min(FACTOR * int(np.ceil(np.log(L))), L)
    scale = float(1.0 / np.sqrt(HIDDEN // H))

    qp, kp, vp = _project(q.reshape(B * L, D), k.reshape(B * L, D),
                          v.reshape(B * L, D), Wq, Wk, Wv)
    qh = qp.reshape(B * H, L, E)
    kh = kp.reshape(B * H, L, E)
    vh = vp.reshape(B * H, L, E)

    ct = _transpose(_build_counts(sample_idx, L, u), L)
    ctx = _attention(qh, kh, vh, ct, L, u, E, scale)
    return ctx.reshape(B, L, HIDDEN)
